# Initial kernel scaffold; baseline (speedup 1.0000x reference)
import jax
import jax.numpy as jnp
from jax.experimental import pallas as pl


def kernel(x, emb):
    raise NotImplementedError("write your pallas kernel here")


# speedup vs baseline: 3.2767x; 3.2767x over previous
import functools

import jax
import jax.numpy as jnp
from jax import lax
from jax.experimental import pallas as pl
from jax.experimental.pallas import tpu as pltpu
from jax.experimental.pallas import tpu_sc as plsc

D_MODEL = 128


@functools.partial(jax.jit, static_argnums=())
def _sc_gather(table, idx_flat):
    B = idx_flat.shape[0]
    D = table.shape[1]
    info = plsc.get_sparse_core_info()
    NC, NS = info.num_cores, info.num_subcores
    NW = NC * NS
    b_per_w = B // NW
    CH = 640
    n_chunks = b_per_w // CH

    mesh = plsc.VectorSubcoreMesh(core_axis_name="c", subcore_axis_name="s")

    @functools.partial(
        pl.kernel,
        mesh=mesh,
        out_type=jax.ShapeDtypeStruct((B, D), jnp.float32),
        scratch_types=[
            pltpu.VMEM((CH,), jnp.int32),
            pltpu.VMEM((CH, D), jnp.float32),
            pltpu.SemaphoreType.DMA,
        ],
    )
    def k(table_hbm, idx_hbm, out_hbm, idx_v, rows_v, sem):
        wid = lax.axis_index("s") * NC + lax.axis_index("c")
        base = wid * b_per_w

        def body(i, carry):
            off = base + i * CH
            pltpu.sync_copy(idx_hbm.at[pl.ds(off, CH)], idx_v)
            pltpu.async_copy(table_hbm.at[idx_v], rows_v, sem).wait()
            pltpu.sync_copy(rows_v, out_hbm.at[pl.ds(off, CH)])
            return carry

        lax.fori_loop(0, n_chunks, body, 0)

    return k(table, idx_flat)


def kernel(x, emb):
    S0, S1 = x.shape
    out = _sc_gather(emb, x.reshape(S0 * S1).astype(jnp.int32))
    return out.reshape(S0, S1, emb.shape[1])


# speedup vs baseline: 3.3222x; 1.0139x over previous
import functools

import jax
import jax.numpy as jnp
from jax import lax
from jax.experimental import pallas as pl
from jax.experimental.pallas import tpu as pltpu
from jax.experimental.pallas import tpu_sc as plsc


def _sc_gather(table, idx_flat):
    B = idx_flat.shape[0]
    D = table.shape[1]
    info = plsc.get_sparse_core_info()
    NC, NS = info.num_cores, info.num_subcores
    NW = NC * NS
    b_per_w = B // NW
    CH = 400
    n_chunks = b_per_w // CH
    n_steps = n_chunks // 2

    mesh = plsc.VectorSubcoreMesh(core_axis_name="c", subcore_axis_name="s")

    @functools.partial(
        pl.kernel,
        mesh=mesh,
        out_type=jax.ShapeDtypeStruct((B, D), jnp.float32),
        scratch_types=[
            pltpu.VMEM((b_per_w,), jnp.int32),
            pltpu.VMEM((CH, D), jnp.float32),
            pltpu.VMEM((CH, D), jnp.float32),
            pltpu.SemaphoreType.DMA,
            pltpu.SemaphoreType.DMA,
            pltpu.SemaphoreType.DMA,
            pltpu.SemaphoreType.DMA,
        ],
    )
    def k(table_hbm, idx_hbm, out_hbm, idx_v, rows0, rows1, g0sem, g1sem,
          w0sem, w1sem):
        wid = lax.axis_index("s") * NC + lax.axis_index("c")
        base = wid * b_per_w
        rows = (rows0, rows1)
        gsem = (g0sem, g1sem)
        wsem = (w0sem, w1sem)

        pltpu.sync_copy(idx_hbm.at[pl.ds(base, b_per_w)], idx_v)

        def start_gather(g, b):
            pltpu.make_async_copy(
                table_hbm.at[idx_v.at[pl.ds(g * CH, CH)]], rows[b], gsem[b]
            ).start()

        def wait_gather(g, b):
            pltpu.make_async_copy(
                table_hbm.at[idx_v.at[pl.ds(g * CH, CH)]], rows[b], gsem[b]
            ).wait()

        def start_write(g, b):
            pltpu.make_async_copy(
                rows[b], out_hbm.at[pl.ds(base + g * CH, CH)], wsem[b]
            ).start()

        def wait_write(g, b):
            pltpu.make_async_copy(
                rows[b], out_hbm.at[pl.ds(base + g * CH, CH)], wsem[b]
            ).wait()

        start_gather(0, 0)

        def body(s, carry):
            g0 = 2 * s
            wait_gather(g0, 0)

            @pl.when(s > 0)
            def _():
                wait_write(g0 - 1, 1)

            start_gather(g0 + 1, 1)
            start_write(g0, 0)
            wait_gather(g0 + 1, 1)

            @pl.when(s < n_steps - 1)
            def _():
                wait_write(g0, 0)
                start_gather(g0 + 2, 0)

            start_write(g0 + 1, 1)
            return carry

        lax.fori_loop(0, n_steps, body, 0)
        wait_write(n_chunks - 2, 0)
        wait_write(n_chunks - 1, 1)

    return k(table, idx_flat)


def kernel(x, emb):
    S0, S1 = x.shape
    out = _sc_gather(emb, x.reshape(S0 * S1).astype(jnp.int32))
    return out.reshape(S0, S1, emb.shape[1])


# speedup vs baseline: 10.2115x; 3.0737x over previous
import functools

import jax
import jax.numpy as jnp
from jax import lax
from jax.experimental import pallas as pl
from jax.experimental.pallas import tpu as pltpu
from jax.experimental.pallas import tpu_sc as plsc


def _sc_gather(table, idx_flat):
    B = idx_flat.shape[0]
    D = table.shape[1]
    info = plsc.get_sparse_core_info()
    NC, NS = info.num_cores, info.num_subcores
    NW = NC * NS
    b_per_w = B // NW
    CH = 400
    n_chunks = b_per_w // CH
    n_steps = n_chunks // 2

    mesh = plsc.VectorSubcoreMesh(core_axis_name="c", subcore_axis_name="s")

    @functools.partial(
        pl.kernel,
        mesh=mesh,
        out_type=jax.ShapeDtypeStruct((B, D), jnp.float32),
        scratch_types=[
            pltpu.VMEM((b_per_w,), jnp.int32),
            pltpu.VMEM((CH, D), jnp.float32),
            pltpu.VMEM((CH, D), jnp.float32),
            pltpu.SemaphoreType.DMA,
            pltpu.SemaphoreType.DMA,
            pltpu.SemaphoreType.DMA,
            pltpu.SemaphoreType.DMA,
        ],
    )
    def k(table_hbm, idx_hbm, out_hbm, idx_v, rows0, rows1, g0sem, g1sem,
          w0sem, w1sem):
        wid = lax.axis_index("s") * NC + lax.axis_index("c")
        base = wid * b_per_w
        rows = (rows0, rows1)
        gsem = (g0sem, g1sem)
        wsem = (w0sem, w1sem)

        pltpu.sync_copy(idx_hbm.at[pl.ds(base, b_per_w)], idx_v)

        def start_gather(g, b):
            pltpu.make_async_copy(
                table_hbm.at[idx_v.at[pl.ds(g * CH, CH)]], rows[b], gsem[b]
            ).start()

        def wait_gather(g, b):
            pltpu.make_async_copy(
                table_hbm.at[idx_v.at[pl.ds(g * CH, CH)]], rows[b], gsem[b]
            ).wait()

        def start_write(g, b):
            pltpu.make_async_copy(
                rows[b], out_hbm.at[pl.ds(base + g * CH, CH)], wsem[b]
            ).start()

        def wait_write(g, b):
            pltpu.make_async_copy(
                rows[b], out_hbm.at[pl.ds(base + g * CH, CH)], wsem[b]
            ).wait()

        start_gather(0, 0)

        def body(s, carry):
            g0 = 2 * s
            wait_gather(g0, 0)

            @pl.when(s > 0)
            def _():
                wait_write(g0 - 1, 1)

            start_gather(g0 + 1, 1)
            start_write(g0, 0)
            wait_gather(g0 + 1, 1)

            @pl.when(s < n_steps - 1)
            def _():
                wait_write(g0, 0)
                start_gather(g0 + 2, 0)

            start_write(g0 + 1, 1)
            return carry

        lax.fori_loop(0, n_steps, body, 0)
        wait_write(n_chunks - 2, 0)
        wait_write(n_chunks - 1, 1)

    return k(table, idx_flat)


def kernel(x, emb):
    S0, S1 = x.shape
    idx_t = x.T.reshape(S0 * S1).astype(jnp.int32)
    out = _sc_gather(emb, idx_t)
    return out.reshape(S1, S0, emb.shape[1]).transpose(1, 0, 2)


# speedup vs baseline: 10.4366x; 1.0220x over previous
import functools

import jax
import jax.numpy as jnp
from jax import lax
from jax.experimental import pallas as pl
from jax.experimental.pallas import tpu as pltpu
from jax.experimental.pallas import tpu_sc as plsc

NBUF = 4
CH = 200


def _sc_gather(table, idx_flat):
    B = idx_flat.shape[0]
    D = table.shape[1]
    info = plsc.get_sparse_core_info()
    NC, NS = info.num_cores, info.num_subcores
    NW = NC * NS
    b_per_w = B // NW
    n_chunks = b_per_w // CH
    n_steps = n_chunks // NBUF
    assert n_chunks % NBUF == 0 and b_per_w % CH == 0

    mesh = plsc.VectorSubcoreMesh(core_axis_name="c", subcore_axis_name="s")

    @functools.partial(
        pl.kernel,
        mesh=mesh,
        out_type=jax.ShapeDtypeStruct((B, D), jnp.float32),
        scratch_types=[
            pltpu.VMEM((b_per_w,), jnp.int32),
            *[pltpu.VMEM((CH, D), jnp.float32) for _ in range(NBUF)],
            *[pltpu.SemaphoreType.DMA for _ in range(2 * NBUF)],
        ],
    )
    def k(table_hbm, idx_hbm, out_hbm, idx_v, *bufs_and_sems):
        rows = bufs_and_sems[:NBUF]
        gsem = bufs_and_sems[NBUF:2 * NBUF]
        wsem = bufs_and_sems[2 * NBUF:]
        wid = lax.axis_index("s") * NC + lax.axis_index("c")
        base = wid * b_per_w

        pltpu.sync_copy(idx_hbm.at[pl.ds(base, b_per_w)], idx_v)

        def gather_copy(g, b):
            return pltpu.make_async_copy(
                table_hbm.at[idx_v.at[pl.ds(g * CH, CH)]], rows[b], gsem[b]
            )

        def write_copy(g, b):
            return pltpu.make_async_copy(
                rows[b], out_hbm.at[pl.ds(base + g * CH, CH)], wsem[b]
            )

        for j in range(NBUF - 1):
            gather_copy(j, j).start()

        def body(s, carry):
            for j in range(NBUF):
                g = s * NBUF + j
                gather_copy(g, j).wait()
                write_copy(g, j).start()
                nxt = g + NBUF - 1
                if j == 0:
                    b_nxt = NBUF - 1

                    @pl.when(s > 0)
                    def _():
                        write_copy(nxt - NBUF, b_nxt).wait()

                    @pl.when(nxt < n_chunks)
                    def _():
                        gather_copy(nxt, b_nxt).start()
                else:
                    b_nxt = j - 1

                    @pl.when(nxt < n_chunks)
                    def _():
                        write_copy(nxt - NBUF, b_nxt).wait()
                        gather_copy(nxt, b_nxt).start()
            return carry

        lax.fori_loop(0, n_steps, body, 0)
        for j in range(NBUF):
            write_copy(n_chunks - NBUF + j, j).wait()

    return k(table, idx_flat)


def kernel(x, emb):
    S0, S1 = x.shape
    idx_t = x.T.reshape(S0 * S1).astype(jnp.int32)
    out = _sc_gather(emb, idx_t)
    return out.reshape(S1, S0, emb.shape[1]).transpose(1, 0, 2)


# speedup vs baseline: 10.5179x; 1.0078x over previous
import functools

import jax
import jax.numpy as jnp
from jax import lax
from jax.experimental import pallas as pl
from jax.experimental.pallas import tpu as pltpu
from jax.experimental.pallas import tpu_sc as plsc

NBUF = 8
CH = 80


def _sc_gather(table, idx_flat):
    B = idx_flat.shape[0]
    D = table.shape[1]
    info = plsc.get_sparse_core_info()
    NC, NS = info.num_cores, info.num_subcores
    NW = NC * NS
    b_per_w = B // NW
    n_chunks = b_per_w // CH
    n_steps = n_chunks // NBUF
    assert n_chunks % NBUF == 0 and b_per_w % CH == 0

    mesh = plsc.VectorSubcoreMesh(core_axis_name="c", subcore_axis_name="s")

    @functools.partial(
        pl.kernel,
        mesh=mesh,
        out_type=jax.ShapeDtypeStruct((B, D), jnp.float32),
        scratch_types=[
            pltpu.VMEM((b_per_w,), jnp.int32),
            *[pltpu.VMEM((CH, D), jnp.float32) for _ in range(NBUF)],
            *[pltpu.SemaphoreType.DMA for _ in range(2 * NBUF)],
        ],
    )
    def k(table_hbm, idx_hbm, out_hbm, idx_v, *bufs_and_sems):
        rows = bufs_and_sems[:NBUF]
        gsem = bufs_and_sems[NBUF:2 * NBUF]
        wsem = bufs_and_sems[2 * NBUF:]
        wid = lax.axis_index("s") * NC + lax.axis_index("c")
        base = wid * b_per_w

        pltpu.sync_copy(idx_hbm.at[pl.ds(base, b_per_w)], idx_v)

        def gather_copy(g, b):
            return pltpu.make_async_copy(
                table_hbm.at[idx_v.at[pl.ds(g * CH, CH)]], rows[b], gsem[b]
            )

        def write_copy(g, b):
            return pltpu.make_async_copy(
                rows[b], out_hbm.at[pl.ds(base + g * CH, CH)], wsem[b]
            )

        for j in range(NBUF - 1):
            gather_copy(j, j).start()

        def body(s, carry):
            for j in range(NBUF):
                g = s * NBUF + j
                gather_copy(g, j).wait()
                write_copy(g, j).start()
                nxt = g + NBUF - 1
                if j == 0:
                    b_nxt = NBUF - 1

                    @pl.when(s > 0)
                    def _():
                        write_copy(nxt - NBUF, b_nxt).wait()

                    @pl.when(nxt < n_chunks)
                    def _():
                        gather_copy(nxt, b_nxt).start()
                else:
                    b_nxt = j - 1

                    @pl.when(nxt < n_chunks)
                    def _():
                        write_copy(nxt - NBUF, b_nxt).wait()
                        gather_copy(nxt, b_nxt).start()
            return carry

        lax.fori_loop(0, n_steps, body, 0)
        for j in range(NBUF):
            write_copy(n_chunks - NBUF + j, j).wait()

    return k(table, idx_flat)


def kernel(x, emb):
    S0, S1 = x.shape
    idx_t = x.T.reshape(S0 * S1).astype(jnp.int32)
    out = _sc_gather(emb, idx_t)
    return out.reshape(S1, S0, emb.shape[1]).transpose(1, 0, 2)


# speedup vs baseline: 10.6792x; 1.0153x over previous
import functools

import jax
import jax.numpy as jnp
from jax import lax
from jax.experimental import pallas as pl
from jax.experimental.pallas import tpu as pltpu
from jax.experimental.pallas import tpu_sc as plsc

CH = 40
VIA_A = [1, 2, 3, 5, 6, 7]
VIA_B = [9, 10, 11, 13, 14, 15]
DIR_A = [0, 4]
DIR_B = [8, 12]
BODY = 16


def _sc_gather(table, idx_flat):
    B = idx_flat.shape[0]
    D = table.shape[1]
    info = plsc.get_sparse_core_info()
    NC, NS = info.num_cores, info.num_subcores
    NW = NC * NS
    b_per_w = B // NW
    n_chunks = b_per_w // CH
    n_steps = n_chunks // BODY
    assert b_per_w % CH == 0 and n_chunks % BODY == 0

    mesh = plsc.VectorSubcoreMesh(core_axis_name="c", subcore_axis_name="s")

    @functools.partial(
        pl.kernel,
        mesh=mesh,
        out_type=jax.ShapeDtypeStruct((B, D), jnp.float32),
        scratch_types=[
            pltpu.VMEM((b_per_w,), jnp.int32),
            pltpu.VMEM_SHARED((NS, 6, CH, D), jnp.float32),
            *[pltpu.VMEM((CH, D), jnp.float32) for _ in range(10)],
            *[pltpu.SemaphoreType.DMA for _ in range(10)],
            *[pltpu.SemaphoreType.DMA for _ in range(6)],
            *[pltpu.SemaphoreType.DMA for _ in range(6)],
            *[pltpu.SemaphoreType.DMA for _ in range(4)],
        ],
    )
    def k(table_hbm, idx_hbm, out_hbm, idx_v, spm, *rest):
        v = rest[0:6]
        d = rest[6:10]
        gv = rest[10:16]
        gd = rest[16:20]
        xsem = rest[20:26]
        dsem = rest[26:32]
        wsem = rest[32:36]
        sid = lax.axis_index("s")
        wid = sid * NC + lax.axis_index("c")
        base = wid * b_per_w

        pltpu.sync_copy(idx_hbm.at[pl.ds(base, b_per_w)], idx_v)

        def gather_copy(g, buf, sem):
            return pltpu.make_async_copy(
                table_hbm.at[idx_v.at[pl.ds(g * CH, CH)]], buf, sem
            )

        def write_copy(g, buf, sem):
            return pltpu.make_async_copy(
                buf, out_hbm.at[pl.ds(base + g * CH, CH)], sem
            )

        def xbar_copy(k_, buf):
            return pltpu.make_async_copy(buf, spm.at[sid, k_], xsem[k_])

        def drain_copy(g, k_):
            return pltpu.make_async_copy(
                spm.at[sid, k_], out_hbm.at[pl.ds(base + g * CH, CH)], dsem[k_]
            )

        for k_, j in enumerate(VIA_A):
            gather_copy(j, v[k_], gv[k_]).start()
        for l_, j in enumerate(DIR_A + DIR_B):
            gather_copy(j, d[l_], gd[l_]).start()

        def body(s, carry):
            g0 = s * BODY
            for k_, j in enumerate(VIA_A):
                g = g0 + j
                gather_copy(g, v[k_], gv[k_]).wait()

                @pl.when(s > 0)
                def _():
                    drain_copy(g, k_).wait()

                xbar_copy(k_, v[k_]).start()
                xbar_copy(k_, v[k_]).wait()
                drain_copy(g, k_).start()
                gather_copy(g + 8, v[k_], gv[k_]).start()
            for l_, j in enumerate(DIR_A):
                g = g0 + j
                gather_copy(g, d[l_], gd[l_]).wait()
                write_copy(g, d[l_], wsem[l_]).start()
            for k_, j in enumerate(VIA_B):
                g = g0 + j
                gather_copy(g, v[k_], gv[k_]).wait()
                drain_copy(g, k_).wait()
                xbar_copy(k_, v[k_]).start()
                xbar_copy(k_, v[k_]).wait()
                drain_copy(g, k_).start()

                @pl.when(s < n_steps - 1)
                def _():
                    gather_copy(g0 + BODY + VIA_A[k_], v[k_], gv[k_]).start()
            for l_, j in enumerate(DIR_B):
                g = g0 + j
                gather_copy(g, d[2 + l_], gd[2 + l_]).wait()
                write_copy(g, d[2 + l_], wsem[2 + l_]).start()
            for l_, j in enumerate(DIR_A + DIR_B):
                write_copy(g0 + j, d[l_], wsem[l_]).wait()

                @pl.when(s < n_steps - 1)
                def _():
                    gather_copy(g0 + BODY + j, d[l_], gd[l_]).start()
            return carry

        lax.fori_loop(0, n_steps, body, 0)
        for k_, j in enumerate(VIA_B):
            drain_copy((n_steps - 1) * BODY + j, k_).wait()

    return k(table, idx_flat)


def kernel(x, emb):
    S0, S1 = x.shape
    idx_t = x.T.reshape(S0 * S1).astype(jnp.int32)
    out = _sc_gather(emb, idx_t)
    return out.reshape(S1, S0, emb.shape[1]).transpose(1, 0, 2)
